# Initial kernel scaffold; baseline (speedup 1.0000x reference)
import jax
import jax.numpy as jnp
from jax.experimental import pallas as pl


def kernel(p, feat, pc, node_tag, params):
    raise NotImplementedError("write your pallas kernel here")


# speedup vs baseline: 13.6299x; 13.6299x over previous
import functools

import jax
import jax.numpy as jnp
from jax import lax
from jax.experimental import pallas as pl
from jax.experimental.pallas import tpu as pltpu
from jax.experimental.pallas import tpu_sc as plsc

BS, NP, NY = 2, 1024, 1024
C_DIM, HID, NB = 32, 256, 5
N_OBJ = 4
K = 20
NPTS = BS * NY
NTAG = BS * N_OBJ
SEG = NPTS // NTAG
NE = NPTS * K
XPAD = 128


def _knn_body(y_ref, x_ref, out_ref):
    t = pl.program_id(0)
    yb = y_ref[0]
    xb = x_ref[0]
    d = -2.0 * lax.dot_general(yb, xb, (((1,), (1,)), ((), ())),
                               preferred_element_type=jnp.float32)
    d = d + jnp.sum(xb * xb, axis=1)[None, :]
    iota = lax.broadcasted_iota(jnp.int32, (SEG, NY), 1)
    off = (t % 2) * NY
    for k in range(K):
        m = jnp.min(d, axis=1, keepdims=True)
        amin = jnp.min(jnp.where(d == m, iota, NY), axis=1, keepdims=True)
        out_ref[:, k:k + 1] = amin + off
        d = jnp.where(iota == amin, jnp.inf, d)


def _knn_topk(y4, x4):
    return pl.pallas_call(
        _knn_body,
        grid=(NTAG,),
        in_specs=[
            pl.BlockSpec((1, SEG, 4), lambda t: (t, 0, 0)),
            pl.BlockSpec((1, NY, 4), lambda t: (t % 2, 0, 0)),
        ],
        out_specs=pl.BlockSpec((SEG, K), lambda t: (t, 0)),
        out_shape=jax.ShapeDtypeStruct((NPTS, K), jnp.int32),
    )(y4, x4)


def _sc_gather(x_tab, idx_flat):
    info = plsc.get_sparse_core_info()
    nw = info.num_cores * info.num_subcores
    bpw = NE // nw
    nch = 2
    cs = bpw // nch

    mesh = plsc.VectorSubcoreMesh(core_axis_name="c", subcore_axis_name="s")

    @functools.partial(
        pl.kernel,
        mesh=mesh,
        out_type=jax.ShapeDtypeStruct((NE, XPAD), jnp.float32),
        scratch_types=[
            pltpu.VMEM((cs,), jnp.int32),
            pltpu.VMEM((cs, XPAD), jnp.float32),
            pltpu.SemaphoreType.DMA,
        ],
    )
    def gather_k(tab_hbm, idx_hbm, out_hbm, idx_v, rows_v, sem):
        wid = lax.axis_index("s") * info.num_cores + lax.axis_index("c")
        for j in range(nch):
            base = wid * bpw + j * cs
            pltpu.sync_copy(idx_hbm.at[pl.ds(base, cs)], idx_v)
            pltpu.async_copy(tab_hbm.at[idx_v], rows_v, sem).wait()
            pltpu.sync_copy(rows_v, out_hbm.at[pl.ds(base, cs)])

    return gather_k(x_tab, idx_flat)


def _conv1_body(fa_ref, yyf_ref, wx_ref, wy_ref, z_ref, st_ref):
    g = pl.program_id(0)
    z = (jnp.dot(fa_ref[...], wx_ref[...], preferred_element_type=jnp.float32)
         + jnp.dot(yyf_ref[...], wy_ref[...], preferred_element_type=jnp.float32))
    z_ref[...] = z

    @pl.when(g == 0)
    def _():
        st_ref[...] = jnp.zeros_like(st_ref)

    st_ref[0:1, :] += jnp.sum(z, axis=0, keepdims=True)
    st_ref[1:2, :] += jnp.sum(z * z, axis=0, keepdims=True)


def _conv1(fa, yyf, wx, wy):
    return pl.pallas_call(
        _conv1_body,
        grid=(K,),
        in_specs=[
            pl.BlockSpec((NPTS, XPAD), lambda g: (g, 0)),
            pl.BlockSpec((NPTS, 48), lambda g: (0, 0)),
            pl.BlockSpec((XPAD, HID), lambda g: (0, 0)),
            pl.BlockSpec((48, HID), lambda g: (0, 0)),
        ],
        out_specs=[
            pl.BlockSpec((NPTS, HID), lambda g: (g, 0)),
            pl.BlockSpec((8, HID), lambda g: (0, 0)),
        ],
        out_shape=[
            jax.ShapeDtypeStruct((NE, HID), jnp.float32),
            jax.ShapeDtypeStruct((8, HID), jnp.float32),
        ],
    )(fa, yyf, wx, wy)


def _norm_lrelu(z, st_ref, g_ref, b_ref):
    mean = st_ref[0:1, :] * (1.0 / NE)
    var = st_ref[1:2, :] * (1.0 / NE) - mean * mean
    xn = (z - mean) * lax.rsqrt(var + 1e-5) * g_ref[...] + b_ref[...]
    return jnp.where(xn > 0, xn, 0.2 * xn)


def _conv_mid_body(z1_ref, st_ref, g_ref, b_ref, w_ref, z2_ref, st2_ref):
    g = pl.program_id(0)
    h = _norm_lrelu(z1_ref[...], st_ref, g_ref, b_ref)
    z = jnp.dot(h, w_ref[...], preferred_element_type=jnp.float32)
    z2_ref[...] = z

    @pl.when(g == 0)
    def _():
        st2_ref[...] = jnp.zeros_like(st2_ref)

    st2_ref[0:1, :] += jnp.sum(z, axis=0, keepdims=True)
    st2_ref[1:2, :] += jnp.sum(z * z, axis=0, keepdims=True)


def _conv_mid(z1, st1, gama, beta, w):
    return pl.pallas_call(
        _conv_mid_body,
        grid=(K,),
        in_specs=[
            pl.BlockSpec((NPTS, HID), lambda g: (g, 0)),
            pl.BlockSpec((8, HID), lambda g: (0, 0)),
            pl.BlockSpec((1, HID), lambda g: (0, 0)),
            pl.BlockSpec((1, HID), lambda g: (0, 0)),
            pl.BlockSpec((HID, HID), lambda g: (0, 0)),
        ],
        out_specs=[
            pl.BlockSpec((NPTS, HID), lambda g: (g, 0)),
            pl.BlockSpec((8, HID), lambda g: (0, 0)),
        ],
        out_shape=[
            jax.ShapeDtypeStruct((NE, HID), jnp.float32),
            jax.ShapeDtypeStruct((8, HID), jnp.float32),
        ],
    )(z1, st1, gama, beta, w)


def _conv3_body(z2_ref, st_ref, g_ref, b_ref, w_ref, c_ref, st3_ref):
    g = pl.program_id(0)
    h = _norm_lrelu(z2_ref[...], st_ref, g_ref, b_ref)
    z = jnp.dot(h, w_ref[...], preferred_element_type=jnp.float32)

    @pl.when(g == 0)
    def _():
        st3_ref[...] = jnp.zeros_like(st3_ref)
        c_ref[...] = jnp.full_like(c_ref, -jnp.inf)

    st3_ref[0:1, :] += jnp.sum(z, axis=0, keepdims=True)
    st3_ref[1:2, :] += jnp.sum(z * z, axis=0, keepdims=True)
    c_ref[...] = jnp.maximum(c_ref[...], z)


def _conv3(z2, st2, gama, beta, w):
    return pl.pallas_call(
        _conv3_body,
        grid=(K,),
        in_specs=[
            pl.BlockSpec((NPTS, HID), lambda g: (g, 0)),
            pl.BlockSpec((8, HID), lambda g: (0, 0)),
            pl.BlockSpec((1, HID), lambda g: (0, 0)),
            pl.BlockSpec((1, HID), lambda g: (0, 0)),
            pl.BlockSpec((HID, C_DIM), lambda g: (0, 0)),
        ],
        out_specs=[
            pl.BlockSpec((NPTS, C_DIM), lambda g: (0, 0)),
            pl.BlockSpec((8, C_DIM), lambda g: (0, 0)),
        ],
        out_shape=[
            jax.ShapeDtypeStruct((NPTS, C_DIM), jnp.float32),
            jax.ShapeDtypeStruct((8, C_DIM), jnp.float32),
        ],
    )(z2, st2, gama, beta, w)


def _pool_body(c_ref, st_ref, g_ref, b_ref, obj_ref):
    t = pl.program_id(0)
    h = _norm_lrelu(c_ref[...], st_ref, g_ref, b_ref)
    obj_ref[pl.ds(t, 1), :] = jnp.max(h, axis=0, keepdims=True)


def _pool(c_pre, st3, gama, beta):
    return pl.pallas_call(
        _pool_body,
        grid=(NTAG,),
        in_specs=[
            pl.BlockSpec((SEG, C_DIM), lambda t: (t, 0)),
            pl.BlockSpec((8, C_DIM), lambda t: (0, 0)),
            pl.BlockSpec((1, C_DIM), lambda t: (0, 0)),
            pl.BlockSpec((1, C_DIM), lambda t: (0, 0)),
        ],
        out_specs=pl.BlockSpec((NTAG, C_DIM), lambda t: (0, 0)),
        out_shape=jax.ShapeDtypeStruct((NTAG, C_DIM), jnp.float32),
    )(c_pre, st3, gama, beta)


def _dec_body(p_ref, obj_ref, fpw_ref, fpb_ref, fcw_ref, fcb_ref,
              w0_ref, b0_ref, w1_ref, b1_ref, wo_ref, out_ref):
    pid = pl.program_id(0)
    net = (jnp.dot(p_ref[0], fpw_ref[...], preferred_element_type=jnp.float32)
           + fpb_ref[...])
    orow = obj_ref[pl.ds(pid, 1), :]
    ci_all = jnp.dot(orow, fcw_ref[...],
                     preferred_element_type=jnp.float32) + fcb_ref[...]
    for i in range(NB):
        net = net + ci_all[:, i * HID:(i + 1) * HID]
        h0 = (jnp.dot(jnp.maximum(net, 0.0), w0_ref[i],
                      preferred_element_type=jnp.float32) + b0_ref[i])
        net = net + (jnp.dot(jnp.maximum(h0, 0.0), w1_ref[i],
                             preferred_element_type=jnp.float32) + b1_ref[i])
    out_ref[pl.ds(pid, 1), :] = lax.dot_general(
        wo_ref[...], jnp.maximum(net, 0.0), (((1,), (1,)), ((), ())),
        preferred_element_type=jnp.float32)


def _decoder(p4, obj, fpw, fpb, fcw, fcb, w0, b0, w1, b1, wo):
    return pl.pallas_call(
        _dec_body,
        grid=(BS * N_OBJ,),
        in_specs=[
            pl.BlockSpec((1, NP, 4), lambda g: (g // N_OBJ, 0, 0)),
            pl.BlockSpec((NTAG, C_DIM), lambda g: (0, 0)),
            pl.BlockSpec((4, HID), lambda g: (0, 0)),
            pl.BlockSpec((1, HID), lambda g: (0, 0)),
            pl.BlockSpec((C_DIM, NB * HID), lambda g: (0, 0)),
            pl.BlockSpec((1, NB * HID), lambda g: (0, 0)),
            pl.BlockSpec((NB, HID, HID), lambda g: (0, 0, 0)),
            pl.BlockSpec((NB, 1, HID), lambda g: (0, 0, 0)),
            pl.BlockSpec((NB, HID, HID), lambda g: (0, 0, 0)),
            pl.BlockSpec((NB, 1, HID), lambda g: (0, 0, 0)),
            pl.BlockSpec((1, HID), lambda g: (0, 0)),
        ],
        out_specs=pl.BlockSpec((BS * N_OBJ, NP), lambda g: (0, 0)),
        out_shape=jax.ShapeDtypeStruct((BS * N_OBJ, NP), jnp.float32),
    )(p4, obj, fpw, fpb, fcw, fcb, w0, b0, w1, b1, wo)


def kernel(p, feat, pc, node_tag, params):
    f32 = jnp.float32
    x = p.reshape(NPTS, 3).astype(f32)
    y = pc.reshape(NPTS, 3).astype(f32)
    yf = feat.reshape(NPTS, C_DIM).astype(f32)

    pad4 = lambda a: jnp.concatenate([a, jnp.zeros((NPTS, 1), f32)], axis=1)
    y4 = pad4(y).reshape(NTAG, SEG, 4)
    x4 = pad4(x).reshape(2, NY, 4)
    nn = _knn_topk(y4, x4)

    x_tab = jnp.concatenate([x, jnp.zeros((NPTS, XPAD - 3), f32)], axis=1)
    fa = _sc_gather(x_tab, nn.T.reshape(-1))

    w1 = params['conv1_w']
    wx = jnp.zeros((XPAD, HID), f32).at[0:3, :].set((w1[:, 3:6] - w1[:, 0:3]).T)
    wy = jnp.zeros((48, HID), f32)
    wy = wy.at[0:3, :].set(w1[:, 0:3].T)
    wy = wy.at[3:3 + C_DIM, :].set(w1[:, 6:6 + C_DIM].T)
    yyf = jnp.zeros((NPTS, 48), f32)
    yyf = yyf.at[:, 0:3].set(y).at[:, 3:3 + C_DIM].set(yf)

    z1, st1 = _conv1(fa, yyf, wx, wy)
    z2, st2 = _conv_mid(z1, st1,
                        params['bn1_g'].reshape(1, HID),
                        params['bn1_b'].reshape(1, HID),
                        params['conv2_w'].T)
    c_pre, st3 = _conv3(z2, st2,
                        params['bn2_g'].reshape(1, HID),
                        params['bn2_b'].reshape(1, HID),
                        params['conv3_w'].T)
    obj = _pool(c_pre, st3,
                params['bn3_g'].reshape(1, C_DIM),
                params['bn3_b'].reshape(1, C_DIM))

    p4 = jnp.concatenate([p.astype(f32),
                          jnp.zeros((BS, NP, 1), f32)], axis=2)
    fpw = jnp.zeros((4, HID), f32).at[0:3, :].set(params['fc_p_w'].T)
    fcw = jnp.transpose(params['fc_c_w'], (2, 0, 1)).reshape(C_DIM, NB * HID)
    fcb = params['fc_c_b'].reshape(1, NB * HID)
    w0 = jnp.transpose(params['blk0_w'], (0, 2, 1))
    b0 = params['blk0_b'].reshape(NB, 1, HID)
    w1t = jnp.transpose(params['blk1_w'], (0, 2, 1))
    b1 = params['blk1_b'].reshape(NB, 1, HID)
    wo = params['fc_out_w'].reshape(1, HID)
    out = _decoder(p4, obj, fpw, params['fc_p_b'].reshape(1, HID),
                   fcw, fcb, w0, b0, w1t, b1, wo)
    out = out + params['fc_out_b'][0]
    return out.reshape(BS, N_OBJ, NP)
